# baseline (speedup 1.0000x reference)
import functools

import jax
import jax.numpy as jnp
from jax import lax
from jax.experimental import pallas as pl
from jax.experimental.pallas import tpu as pltpu
from jax.experimental.pallas import tpu_sc as plsc

NUM_HEADS = 16
VOCAB = 512
B_TOTAL = 8 * 512 * 512
NW = 32
B_W = B_TOTAL // NW
CHUNK = 1024
N_CHUNKS = B_W // CHUNK
NBUF = 4
N_ROUNDS = N_CHUNKS // NBUF

_mesh = plsc.VectorSubcoreMesh(core_axis_name="c", subcore_axis_name="s")


@functools.partial(
    pl.kernel,
    mesh=_mesh,
    out_type=jax.ShapeDtypeStruct((B_TOTAL, NUM_HEADS), jnp.float32),
    scratch_types=[
        pltpu.VMEM((NBUF, CHUNK), jnp.int32),
        pltpu.VMEM((NBUF, CHUNK, NUM_HEADS), jnp.float32),
        pltpu.VMEM((VOCAB, NUM_HEADS), jnp.float32),
        pltpu.SemaphoreType.DMA((NBUF,)),
        pltpu.SemaphoreType.DMA((NBUF,)),
    ],
    compiler_params=pltpu.CompilerParams(use_tc_tiling_on_sc=False,
                                         needs_layout_passes=False),
)
def _gather_kernel(table_hbm, idx_hbm, out_hbm, idx_v, rows_v, table_v,
                   idx_sem, wb_sem):
    sid = lax.axis_index("s")
    wid = sid * 2 + lax.axis_index("c")
    base = wid * B_W

    pltpu.sync_copy(table_hbm, table_v)

    def start_idx(c, b):
        pltpu.async_copy(idx_hbm.at[pl.ds(base + c * CHUNK, CHUNK)],
                         idx_v.at[b], idx_sem.at[b])

    def wait_idx(b):
        pltpu.make_async_copy(idx_hbm.at[pl.ds(base, CHUNK)],
                              idx_v.at[b], idx_sem.at[b]).wait()

    def start_wb(c, b):
        pltpu.async_copy(rows_v.at[b],
                         out_hbm.at[pl.ds(base + c * CHUNK, CHUNK)],
                         wb_sem.at[b])

    def wait_wb(b):
        pltpu.make_async_copy(rows_v.at[b],
                              out_hbm.at[pl.ds(base, CHUNK)],
                              wb_sem.at[b]).wait()

    iota16 = lax.broadcasted_iota(jnp.int32, (16,), 0)
    cols = [jnp.full((16,), j, jnp.int32) for j in range(NUM_HEADS)]

    def compute_chunk(b):
        def body(i, carry):
            i0 = i * 16
            idx_vec = idx_v[b, pl.ds(i0, 16)]
            row_ids = i0 + iota16
            for j in range(NUM_HEADS):
                vals = plsc.load_gather(table_v, [idx_vec, cols[j]])
                plsc.store_scatter(rows_v.at[b], [row_ids, cols[j]], vals)
            return carry
        lax.fori_loop(0, CHUNK // 16, body, 0, unroll=2)

    for b in range(NBUF):
        start_idx(b, b)

    for b in range(NBUF):
        wait_idx(b)
        compute_chunk(b)
        start_idx(b + NBUF, b)
        start_wb(b, b)

    def round_body(r, carry):
        g0 = r * NBUF
        for b in range(NBUF):
            wait_wb(b)
            wait_idx(b)
            compute_chunk(b)
            start_idx(g0 + b + NBUF, b)
            start_wb(g0 + b, b)
        return carry

    lax.fori_loop(1, N_ROUNDS - 1, round_body, 0)

    g0 = (N_ROUNDS - 1) * NBUF
    for b in range(NBUF):
        wait_wb(b)
        wait_idx(b)
        compute_chunk(b)
        start_wb(g0 + b, b)

    for b in range(NBUF):
        wait_wb(b)


def kernel(dist, embedding_table):
    idx = dist.reshape(-1).astype(jnp.int32)
    out = _gather_kernel(embedding_table, idx)
    return out.reshape(*dist.shape, NUM_HEADS)


# speedup vs baseline: 1.6150x; 1.6150x over previous
import functools

import jax
import jax.numpy as jnp
from jax import lax
from jax.experimental import pallas as pl
from jax.experimental.pallas import tpu as pltpu
from jax.experimental.pallas import tpu_sc as plsc

NUM_HEADS = 16
VOCAB = 512
B_TOTAL = 8 * 512 * 512
NW = 32
B_W = B_TOTAL // NW
CHUNK = 1024
N_CHUNKS = B_W // CHUNK
NBUF = 4
N_ROUNDS = N_CHUNKS // NBUF

_mesh = plsc.VectorSubcoreMesh(core_axis_name="c", subcore_axis_name="s")


@functools.partial(
    pl.kernel,
    mesh=_mesh,
    out_type=jax.ShapeDtypeStruct((B_TOTAL, NUM_HEADS), jnp.float32),
    scratch_types=[
        pltpu.VMEM((NBUF, CHUNK), jnp.int32),
        pltpu.VMEM((NBUF, CHUNK, NUM_HEADS), jnp.float32),
        pltpu.VMEM_SHARED((VOCAB, NUM_HEADS), jnp.float32),
        pltpu.SemaphoreType.DMA((NBUF,)),
        pltpu.SemaphoreType.DMA((NBUF,)),
        pltpu.SemaphoreType.DMA((NBUF,)),
    ],
    compiler_params=pltpu.CompilerParams(use_tc_tiling_on_sc=False),
)
def _gather_kernel(table_hbm, idx_hbm, out_hbm, idx_v, rows_v, table_sh,
                   idx_sem, gat_sem, wb_sem):
    sid = lax.axis_index("s")
    wid = sid * 2 + lax.axis_index("c")
    base = wid * B_W

    @pl.when(sid == 0)
    def _stage_table():
        pltpu.sync_copy(table_hbm, table_sh)

    plsc.subcore_barrier()

    def start_idx(c, b):
        pltpu.async_copy(idx_hbm.at[pl.ds(base + c * CHUNK, CHUNK)],
                         idx_v.at[b], idx_sem.at[b])

    def wait_idx(b):
        pltpu.make_async_copy(idx_hbm.at[pl.ds(base, CHUNK)],
                              idx_v.at[b], idx_sem.at[b]).wait()

    def start_gather(b):
        pltpu.async_copy(table_sh.at[idx_v.at[b]], rows_v.at[b],
                         gat_sem.at[b])

    def wait_gather(b):
        pltpu.make_async_copy(table_sh.at[idx_v.at[b]], rows_v.at[b],
                              gat_sem.at[b]).wait()

    def start_wb(c, b):
        pltpu.async_copy(rows_v.at[b],
                         out_hbm.at[pl.ds(base + c * CHUNK, CHUNK)],
                         wb_sem.at[b])

    def wait_wb(b):
        pltpu.make_async_copy(rows_v.at[b],
                              out_hbm.at[pl.ds(base, CHUNK)],
                              wb_sem.at[b]).wait()

    for b in range(NBUF):
        start_idx(b, b)

    for g in range(NBUF):
        wait_idx(g)
        start_gather(g)
        if g >= 2:
            h = g - 2
            wait_gather(h)
            start_wb(h, h)
            start_idx(h + NBUF, h)

    def round_body(r, carry):
        g0 = r * NBUF
        for b in range(NBUF):
            bh = (b + 2) % NBUF
            wait_wb(b)
            wait_idx(b)
            start_gather(b)
            wait_gather(bh)
            start_wb(g0 + b - 2, bh)
            start_idx(g0 + b + 2, bh)
        return carry

    lax.fori_loop(1, N_ROUNDS - 1, round_body, 0)

    g0 = (N_ROUNDS - 1) * NBUF
    for b in range(NBUF):
        bh = (b + 2) % NBUF
        wait_wb(b)
        wait_idx(b)
        start_gather(b)
        wait_gather(bh)
        start_wb(g0 + b - 2, bh)
        if g0 + b + 2 < N_CHUNKS:
            start_idx(g0 + b + 2, bh)

    for g in (N_CHUNKS - 2, N_CHUNKS - 1):
        b = g % NBUF
        wait_gather(b)
        start_wb(g, b)
    for b in range(NBUF):
        wait_wb(b)


def kernel(dist, embedding_table):
    idx = dist.reshape(-1).astype(jnp.int32)
    out = _gather_kernel(embedding_table, idx)
    return out.reshape(*dist.shape, NUM_HEADS)


# speedup vs baseline: 3.8558x; 2.3874x over previous
import functools

import jax
import jax.numpy as jnp
from jax import lax
from jax.experimental import pallas as pl
from jax.experimental.pallas import tpu as pltpu
from jax.experimental.pallas import tpu_sc as plsc

NUM_HEADS = 16
VOCAB = 512
B_TOTAL = 8 * 512 * 512
NW = 32
CHUNK = 1024
N_CHUNKS = B_TOTAL // CHUNK
CPW = N_CHUNKS // NW

_mesh = plsc.VectorSubcoreMesh(core_axis_name="c", subcore_axis_name="s")


@functools.partial(
    pl.kernel,
    mesh=_mesh,
    out_type=jax.ShapeDtypeStruct((8, 512, 2, 4, 8, 128), jnp.float32),
    scratch_types=[
        pltpu.VMEM((2, CHUNK), jnp.int32),
        pltpu.VMEM((2, CHUNK, NUM_HEADS), jnp.float32),
        pltpu.VMEM((2, 32, 513), jnp.float32),
        pltpu.VMEM_SHARED((VOCAB, NUM_HEADS), jnp.float32),
        pltpu.SemaphoreType.DMA((2,)),
        pltpu.SemaphoreType.DMA((2,)),
        pltpu.SemaphoreType.DMA((2,)),
    ],
    compiler_params=pltpu.CompilerParams(use_tc_tiling_on_sc=False,
                                         needs_layout_passes=False),
)
def _gather_kernel(table_hbm, idx_hbm, out_hbm, idx_v, rows_v, out_pad,
                   table_sh, idx_sem, gat_sem, wb_sem):
    sid = lax.axis_index("s")
    w = sid * 2 + lax.axis_index("c")
    c0 = w * CPW

    @pl.when(sid == 0)
    def _stage_table():
        pltpu.sync_copy(table_hbm, table_sh)

    plsc.subcore_barrier()

    iota16 = lax.broadcasted_iota(jnp.int32, (16,), 0)
    row_ids = [jnp.full((16,), il * 16, jnp.int32) + iota16 for il in range(2)]

    def decode(c):
        return c // 256, c % 256

    def start_idx(c, buf):
        pltpu.async_copy(idx_hbm.at[pl.ds(c * CHUNK, CHUNK)],
                         idx_v.at[buf], idx_sem.at[buf])

    def wait_idx(buf):
        pltpu.make_async_copy(idx_hbm.at[pl.ds(0, CHUNK)],
                              idx_v.at[buf], idx_sem.at[buf]).wait()

    def start_gathers(buf):
        for q in range(8):
            pltpu.async_copy(
                table_sh.at[idx_v.at[buf, pl.ds(q * 128, 128)]],
                rows_v.at[buf, pl.ds(q * 128, 128)], gat_sem.at[buf])

    def wait_gathers(buf):
        for q in range(8):
            pltpu.make_async_copy(
                table_sh.at[idx_v.at[buf, pl.ds(q * 128, 128)]],
                rows_v.at[buf, pl.ds(q * 128, 128)], gat_sem.at[buf]).wait()

    def start_wb(c, buf):
        b, i2 = decode(c)
        for il in range(2):
            for ht in range(2):
                for jt in range(4):
                    pltpu.async_copy(
                        out_pad.at[buf, pl.ds(il * 16 + ht * 8, 8),
                                   pl.ds(jt * 128, 128)],
                        out_hbm.at[b, i2 * 2 + il, ht, jt],
                        wb_sem.at[buf])

    def wait_wb(buf):
        for il in range(2):
            for ht in range(2):
                for jt in range(4):
                    pltpu.make_async_copy(
                        out_pad.at[buf, pl.ds(il * 16 + ht * 8, 8),
                                   pl.ds(jt * 128, 128)],
                        out_hbm.at[0, 0, ht, jt],
                        wb_sem.at[buf]).wait()

    def compute(buf):
        def body(j, col):
            for il in range(2):
                vals = rows_v[buf, il * 512 + j]
                plsc.store_scatter(out_pad.at[buf], [row_ids[il], col], vals)
            return col + 1

        lax.fori_loop(0, 512, body, jnp.zeros((16,), jnp.int32), unroll=4)

    def run_chunk(c, buf, skip_wb_wait, has1, has2):
        if has1:
            wait_idx(1 - buf)
            start_gathers(1 - buf)
        wait_gathers(buf)
        if not skip_wb_wait:
            wait_wb(buf)
        compute(buf)
        start_wb(c, buf)
        if has2:
            start_idx(c + 2, buf)

    start_idx(c0, 0)
    start_idx(c0 + 1, 1)
    wait_idx(0)
    start_gathers(0)
    run_chunk(c0 + 0, 0, True, True, True)
    run_chunk(c0 + 1, 1, True, True, True)

    def rounds(r, carry):
        g = c0 + 2 + 2 * r
        run_chunk(g, 0, False, True, True)
        run_chunk(g + 1, 1, False, True, True)
        return carry

    lax.fori_loop(0, (CPW - 4) // 2, rounds, 0)

    run_chunk(c0 + CPW - 2, 0, False, True, False)
    run_chunk(c0 + CPW - 1, 1, False, False, False)
    wait_wb(0)
    wait_wb(1)


def kernel(dist, embedding_table):
    idx = dist.reshape(-1).astype(jnp.int32)
    out = _gather_kernel(embedding_table, idx)
    return out.transpose(0, 1, 3, 5, 2, 4).reshape(8, 512, 512, NUM_HEADS)


# speedup vs baseline: 8.2749x; 2.1461x over previous
import functools

import jax
import jax.numpy as jnp
from jax import lax
from jax.experimental import pallas as pl
from jax.experimental.pallas import tpu as pltpu
from jax.experimental.pallas import tpu_sc as plsc

NUM_HEADS = 16
VOCAB = 512
B_TOTAL = 8 * 512 * 512
NW = 32
CHUNK = 1024
N_CHUNKS = B_TOTAL // CHUNK
CPW = N_CHUNKS // NW

_mesh = plsc.VectorSubcoreMesh(core_axis_name="c", subcore_axis_name="s")


@functools.partial(
    pl.kernel,
    mesh=_mesh,
    out_type=jax.ShapeDtypeStruct((8, 512, 2, 4, 8, 128), jnp.float32),
    scratch_types=[
        pltpu.VMEM((2, CHUNK), jnp.int32),
        pltpu.VMEM((2, CHUNK, NUM_HEADS), jnp.float32),
        pltpu.VMEM((2, 32, 513), jnp.float32),
        pltpu.VMEM_SHARED((VOCAB, NUM_HEADS), jnp.float32),
        pltpu.SemaphoreType.DMA((2,)),
        pltpu.SemaphoreType.DMA((2,)),
        pltpu.SemaphoreType.DMA((2,)),
    ],
    compiler_params=pltpu.CompilerParams(use_tc_tiling_on_sc=False,
                                         needs_layout_passes=False),
)
def _gather_kernel(table_hbm, idx_hbm, out_hbm, idx_v, rows_v, out_pad,
                   table_sh, idx_sem, gat_sem, wb_sem):
    sid = lax.axis_index("s")
    w = sid * 2 + lax.axis_index("c")
    c0 = w * CPW

    @pl.when(sid == 0)
    def _stage_table():
        pltpu.sync_copy(table_hbm, table_sh)

    plsc.subcore_barrier()

    iota16 = lax.broadcasted_iota(jnp.int32, (16,), 0)
    row_ids = [jnp.full((16,), il * 16, jnp.int32) + iota16 for il in range(2)]

    def decode(c):
        return c // 256, c % 256

    def start_idx(c, buf):
        pltpu.async_copy(idx_hbm.at[pl.ds(c * CHUNK, CHUNK)],
                         idx_v.at[buf], idx_sem.at[buf])

    def wait_idx(buf):
        pltpu.make_async_copy(idx_hbm.at[pl.ds(0, CHUNK)],
                              idx_v.at[buf], idx_sem.at[buf]).wait()

    def start_gathers(buf):
        for q in range(8):
            pltpu.async_copy(
                table_sh.at[idx_v.at[buf, pl.ds(q * 128, 128)]],
                rows_v.at[buf, pl.ds(q * 128, 128)], gat_sem.at[buf])

    def wait_gathers(buf):
        for q in range(8):
            pltpu.make_async_copy(
                table_sh.at[idx_v.at[buf, pl.ds(q * 128, 128)]],
                rows_v.at[buf, pl.ds(q * 128, 128)], gat_sem.at[buf]).wait()

    def start_wb(c, buf):
        b, i2 = decode(c)
        for il in range(2):
            for ht in range(2):
                for jt in range(4):
                    pltpu.async_copy(
                        out_pad.at[buf, pl.ds(il * 16 + ht * 8, 8),
                                   pl.ds(jt * 128, 128)],
                        out_hbm.at[b, i2 * 2 + il, ht, jt],
                        wb_sem.at[buf])

    def wait_wb(buf):
        for il in range(2):
            for ht in range(2):
                for jt in range(4):
                    pltpu.make_async_copy(
                        out_pad.at[buf, pl.ds(il * 16 + ht * 8, 8),
                                   pl.ds(jt * 128, 128)],
                        out_hbm.at[0, 0, ht, jt],
                        wb_sem.at[buf]).wait()

    def compute(buf):
        @plsc.parallel_loop(0, 512, 1, unroll=8)
        def _body(j):
            col = jnp.full((16,), j, jnp.int32)
            for il in range(2):
                vals = rows_v[buf, il * 512 + j]
                plsc.store_scatter(out_pad.at[buf], [row_ids[il], col], vals)

    def run_chunk(c, buf, skip_wb_wait, has1, has2):
        if has1:
            wait_idx(1 - buf)
            start_gathers(1 - buf)
        wait_gathers(buf)
        if not skip_wb_wait:
            wait_wb(buf)
        compute(buf)
        start_wb(c, buf)
        if has2:
            start_idx(c + 2, buf)

    start_idx(c0, 0)
    start_idx(c0 + 1, 1)
    wait_idx(0)
    start_gathers(0)
    run_chunk(c0 + 0, 0, True, True, True)
    run_chunk(c0 + 1, 1, True, True, True)

    def rounds(r, carry):
        g = c0 + 2 + 2 * r
        run_chunk(g, 0, False, True, True)
        run_chunk(g + 1, 1, False, True, True)
        return carry

    lax.fori_loop(0, (CPW - 4) // 2, rounds, 0)

    run_chunk(c0 + CPW - 2, 0, False, True, False)
    run_chunk(c0 + CPW - 1, 1, False, False, False)
    wait_wb(0)
    wait_wb(1)


def kernel(dist, embedding_table):
    idx = dist.reshape(-1).astype(jnp.int32)
    out = _gather_kernel(embedding_table, idx)
    return out.transpose(0, 1, 3, 5, 2, 4).reshape(8, 512, 512, NUM_HEADS)


# speedup vs baseline: 8.3294x; 1.0066x over previous
import functools

import jax
import jax.numpy as jnp
from jax import lax
from jax.experimental import pallas as pl
from jax.experimental.pallas import tpu as pltpu
from jax.experimental.pallas import tpu_sc as plsc

NUM_HEADS = 16
VOCAB = 512
B_TOTAL = 8 * 512 * 512
NW = 32
CHUNK = 1024
N_CHUNKS = B_TOTAL // CHUNK
CPW = N_CHUNKS // NW

_mesh = plsc.VectorSubcoreMesh(core_axis_name="c", subcore_axis_name="s")


@functools.partial(
    pl.kernel,
    mesh=_mesh,
    out_type=jax.ShapeDtypeStruct((8, 512, 2, 4, 8, 128), jnp.float32),
    scratch_types=[
        pltpu.VMEM((2, CHUNK), jnp.int32),
        pltpu.VMEM((2, CHUNK, NUM_HEADS), jnp.float32),
        pltpu.VMEM((2, 32, 513), jnp.float32),
        pltpu.VMEM_SHARED((VOCAB, NUM_HEADS), jnp.float32),
        pltpu.SemaphoreType.DMA((2,)),
        pltpu.SemaphoreType.DMA((2,)),
        pltpu.SemaphoreType.DMA((2,)),
    ],
    compiler_params=pltpu.CompilerParams(use_tc_tiling_on_sc=False,
                                         needs_layout_passes=False),
)
def _gather_kernel(table_hbm, idx_hbm, out_hbm, idx_v, rows_v, out_pad,
                   table_sh, idx_sem, gat_sem, wb_sem):
    sid = lax.axis_index("s")
    w = sid * 2 + lax.axis_index("c")
    c0 = w * CPW

    @pl.when(sid == 0)
    def _stage_table():
        pltpu.sync_copy(table_hbm, table_sh)

    plsc.subcore_barrier()

    iota16 = lax.broadcasted_iota(jnp.int32, (16,), 0)
    row_ids = [jnp.full((16,), il * 16, jnp.int32) + iota16 for il in range(2)]

    def decode(c):
        return c // 256, c % 256

    def start_idx(c, buf):
        pltpu.async_copy(idx_hbm.at[pl.ds(c * CHUNK, CHUNK)],
                         idx_v.at[buf], idx_sem.at[buf])

    def wait_idx(buf):
        pltpu.make_async_copy(idx_hbm.at[pl.ds(0, CHUNK)],
                              idx_v.at[buf], idx_sem.at[buf]).wait()

    def start_gathers(buf):
        pltpu.async_copy(table_sh.at[idx_v.at[buf]], rows_v.at[buf],
                         gat_sem.at[buf])

    def wait_gathers(buf):
        pltpu.make_async_copy(table_sh.at[idx_v.at[buf]], rows_v.at[buf],
                              gat_sem.at[buf]).wait()

    def start_wb(c, buf):
        b, i2 = decode(c)
        for il in range(2):
            for ht in range(2):
                for jt in range(4):
                    pltpu.async_copy(
                        out_pad.at[buf, pl.ds(il * 16 + ht * 8, 8),
                                   pl.ds(jt * 128, 128)],
                        out_hbm.at[b, i2 * 2 + il, ht, jt],
                        wb_sem.at[buf])

    def wait_wb(buf):
        for il in range(2):
            for ht in range(2):
                for jt in range(4):
                    pltpu.make_async_copy(
                        out_pad.at[buf, pl.ds(il * 16 + ht * 8, 8),
                                   pl.ds(jt * 128, 128)],
                        out_hbm.at[0, 0, ht, jt],
                        wb_sem.at[buf]).wait()

    def compute(buf):
        @plsc.parallel_loop(0, 512, 1, unroll=16)
        def _body(j):
            col = jnp.full((16,), j, jnp.int32)
            for il in range(2):
                vals = rows_v[buf, il * 512 + j]
                plsc.store_scatter(out_pad.at[buf], [row_ids[il], col], vals)

    def run_chunk(c, buf, skip_wb_wait, has1, has2):
        if has1:
            wait_idx(1 - buf)
            start_gathers(1 - buf)
        wait_gathers(buf)
        if not skip_wb_wait:
            wait_wb(buf)
        compute(buf)
        start_wb(c, buf)
        if has2:
            start_idx(c + 2, buf)

    start_idx(c0, 0)
    start_idx(c0 + 1, 1)
    wait_idx(0)
    start_gathers(0)
    run_chunk(c0 + 0, 0, True, True, True)
    run_chunk(c0 + 1, 1, True, True, True)

    def rounds(r, carry):
        g = c0 + 2 + 2 * r
        run_chunk(g, 0, False, True, True)
        run_chunk(g + 1, 1, False, True, True)
        return carry

    lax.fori_loop(0, (CPW - 4) // 2, rounds, 0)

    run_chunk(c0 + CPW - 2, 0, False, True, False)
    run_chunk(c0 + CPW - 1, 1, False, False, False)
    wait_wb(0)
    wait_wb(1)


def kernel(dist, embedding_table):
    idx = dist.reshape(-1).astype(jnp.int32)
    out = _gather_kernel(embedding_table, idx)
    return out.transpose(0, 1, 3, 5, 2, 4).reshape(8, 512, 512, NUM_HEADS)
